# Initial kernel scaffold; baseline (speedup 1.0000x reference)
import jax
import jax.numpy as jnp
from jax.experimental import pallas as pl


def kernel(x, edge_index, batch, init_W0, init_b0, init_g0, init_be0, init_W1, init_b1, init_g1, init_be1, mp_W0, mp_b0, mp_g0, mp_be0, mp_W1, mp_b1, mp_g1, mp_be1, head_W0, head_b0, head_W1, head_b1):
    raise NotImplementedError("write your pallas kernel here")


# speedup vs baseline: 3.3807x; 3.3807x over previous
import functools

import jax
import jax.numpy as jnp
from jax import lax
from jax.experimental import pallas as pl
from jax.experimental.pallas import tpu as pltpu
from jax.experimental.pallas import tpu_sc as plsc

N = 10000
E = 160000
D = 256
H = 512
C = 16
G = 64
BN_EPS = 1e-5

R = 1000
NT = N // R
EB = 80
ROWS_PER_TILE = E // 16 // EB
NODES_PER_TILE = N // 16


def _sc_agg_body(npc, xf, xf3, srcm, dstm, sf3, src_v, dst_v, rows_v, spmem, sem):
    ci = lax.axis_index("c")
    si = lax.axis_index("s")
    pltpu.sync_copy(dstm.at[si], dst_v)
    for t in range(npc):
        chunk = ci * npc + t
        pltpu.sync_copy(srcm.at[chunk * 16 + si], src_v)
        pltpu.sync_copy(xf3.at[chunk * 16 + si],
                        spmem.at[pl.ds(si * NODES_PER_TILE, NODES_PER_TILE)])
        plsc.subcore_barrier()

        def eb_step(b, carry):
            pltpu.async_copy(xf.at[src_v.at[b]], rows_v, sem).wait()
            pltpu.sync_copy(rows_v, spmem.at[dst_v.at[b]], add=True)
            return carry

        lax.fori_loop(0, ROWS_PER_TILE, eb_step, 0)
        plsc.subcore_barrier()
        pltpu.sync_copy(spmem.at[pl.ds(si * NODES_PER_TILE, NODES_PER_TILE)],
                        sf3.at[chunk * 16 + si])
        if t + 1 < npc:
            plsc.subcore_barrier()


def _sc_agg(xf, srcm, dstm, nc):
    npc = nc // 2
    mesh = plsc.VectorSubcoreMesh(core_axis_name="c", subcore_axis_name="s")
    xf3 = xf.reshape(nc * 16, NODES_PER_TILE, 128)
    out = pl.kernel(
        functools.partial(_sc_agg_body, npc),
        out_type=jax.ShapeDtypeStruct((nc * 16, NODES_PER_TILE, 128), jnp.float32),
        mesh=mesh,
        scratch_types=[
            pltpu.VMEM((ROWS_PER_TILE, EB), jnp.int32),
            pltpu.VMEM((ROWS_PER_TILE, EB), jnp.int32),
            pltpu.VMEM((EB, 128), jnp.float32),
            pltpu.VMEM_SHARED((N, 128), jnp.float32),
            pltpu.SemaphoreType.DMA,
        ],
    )(xf, xf3, srcm, dstm)
    return out.reshape(nc * N, 128)


def _k1_body(s_ref, w_ref, b_ref, h_ref, st_ref, acc, stacc):
    c = pl.program_id(1)
    nc = pl.num_programs(1)
    i = pl.program_id(0)
    ni = pl.num_programs(0)

    @pl.when(c == 0)
    def _():
        acc[...] = jnp.zeros_like(acc)

    acc[...] += jnp.dot(s_ref[...], w_ref[0],
                        preferred_element_type=jnp.float32)

    @pl.when(c == nc - 1)
    def _():
        h = jnp.maximum(acc[...] + b_ref[...], 0.0)
        h_ref[...] = h

        @pl.when(i == 0)
        def _():
            stacc[...] = jnp.zeros_like(stacc)

        stacc[0:1, :] += jnp.sum(h, axis=0, keepdims=True)
        stacc[1:2, :] += jnp.sum(h * h, axis=0, keepdims=True)

        @pl.when(i == ni - 1)
        def _():
            st_ref[...] = stacc[...]


def _k1(s, w, b, nc):
    return pl.pallas_call(
        _k1_body,
        grid=(NT, nc),
        in_specs=[
            pl.BlockSpec((R, 128), lambda i, c: (c * NT + i, 0)),
            pl.BlockSpec((1, 128, H), lambda i, c: (c, 0, 0)),
            pl.BlockSpec((1, H), lambda i, c: (0, 0)),
        ],
        out_specs=[
            pl.BlockSpec((R, H), lambda i, c: (i, 0)),
            pl.BlockSpec((8, H), lambda i, c: (0, 0)),
        ],
        out_shape=[
            jax.ShapeDtypeStruct((N, H), jnp.float32),
            jax.ShapeDtypeStruct((8, H), jnp.float32),
        ],
        scratch_shapes=[
            pltpu.VMEM((R, H), jnp.float32),
            pltpu.VMEM((8, H), jnp.float32),
        ],
    )(s, w, b)


def _k2_body(h_ref, st_ref, g_ref, be_ref, w_ref, b_ref, out_ref, sto_ref, stacc):
    i = pl.program_id(0)
    ni = pl.num_programs(0)
    mean = st_ref[0:1, :] * (1.0 / N)
    var = st_ref[1:2, :] * (1.0 / N) - mean * mean
    a = g_ref[...] * lax.rsqrt(var + BN_EPS)
    cc = be_ref[...] - mean * a
    hn = h_ref[...] * a + cc
    h2 = jnp.maximum(
        jnp.dot(hn, w_ref[...], preferred_element_type=jnp.float32) + b_ref[...],
        0.0)
    for c in range(4):
        out_ref[c] = h2[:, c * 128:(c + 1) * 128]

    @pl.when(i == 0)
    def _():
        stacc[...] = jnp.zeros_like(stacc)

    stacc[0:1, :] += jnp.sum(h2, axis=0, keepdims=True)
    stacc[1:2, :] += jnp.sum(h2 * h2, axis=0, keepdims=True)

    @pl.when(i == ni - 1)
    def _():
        sto_ref[...] = stacc[...]


def _k2(h, st, g, be, w, b):
    return pl.pallas_call(
        _k2_body,
        grid=(NT,),
        in_specs=[
            pl.BlockSpec((R, H), lambda i: (i, 0)),
            pl.BlockSpec((8, H), lambda i: (0, 0)),
            pl.BlockSpec((1, H), lambda i: (0, 0)),
            pl.BlockSpec((1, H), lambda i: (0, 0)),
            pl.BlockSpec((H, H), lambda i: (0, 0)),
            pl.BlockSpec((1, H), lambda i: (0, 0)),
        ],
        out_specs=[
            pl.BlockSpec((4, R, 128), lambda i: (0, i, 0)),
            pl.BlockSpec((8, H), lambda i: (0, 0)),
        ],
        out_shape=[
            jax.ShapeDtypeStruct((4, N, 128), jnp.float32),
            jax.ShapeDtypeStruct((8, H), jnp.float32),
        ],
        scratch_shapes=[pltpu.VMEM((8, H), jnp.float32)],
    )(h, st, g, be, w, b)


def _k0_body(h_ref, st_ref, g_ref, be_ref, out_ref):
    mean = st_ref[0:1, :] * (1.0 / N)
    var = st_ref[1:2, :] * (1.0 / N) - mean * mean
    a = g_ref[...] * lax.rsqrt(var + BN_EPS)
    cc = be_ref[...] - mean * a
    for c in range(4):
        sl = slice(c * 128, (c + 1) * 128)
        out_ref[c] = h_ref[c] * a[:, sl] + cc[:, sl]


def _k0(hc, st, g, be):
    return pl.pallas_call(
        _k0_body,
        grid=(NT,),
        in_specs=[
            pl.BlockSpec((4, R, 128), lambda i: (0, i, 0)),
            pl.BlockSpec((8, H), lambda i: (0, 0)),
            pl.BlockSpec((1, H), lambda i: (0, 0)),
            pl.BlockSpec((1, H), lambda i: (0, 0)),
        ],
        out_specs=pl.BlockSpec((4, R, 128), lambda i: (0, i, 0)),
        out_shape=jax.ShapeDtypeStruct((4, N, 128), jnp.float32),
    )(hc, st, g, be)


def _k3_body(h_ref, b_ref, w0_ref, b0_ref, w1_ref, b1_ref, out_ref, psum, cnt):
    i = pl.program_id(0)
    ni = pl.num_programs(0)

    @pl.when(i == 0)
    def _():
        psum[...] = jnp.zeros_like(psum)
        cnt[...] = jnp.zeros_like(cnt)

    bt = b_ref[0]
    gids = lax.broadcasted_iota(jnp.int32, (G, 1), 0)
    oh = (bt == gids).astype(jnp.float32)
    for c in range(4):
        psum[c] += jnp.dot(oh, h_ref[c], preferred_element_type=jnp.float32)
    cnt[...] += jnp.sum(oh, axis=1, keepdims=True)

    @pl.when(i == ni - 1)
    def _():
        inv = 1.0 / jnp.maximum(cnt[...], 1.0)
        z1 = b0_ref[...] + jnp.zeros((G, H), jnp.float32)
        for c in range(4):
            z1 += jnp.dot(psum[c] * inv, w0_ref[c * 128:(c + 1) * 128, :],
                          preferred_element_type=jnp.float32)
        z1 = jnp.maximum(z1, 0.0)
        z = jnp.dot(z1, w1_ref[...], preferred_element_type=jnp.float32) + b1_ref[...]
        mx = jnp.max(z, axis=1, keepdims=True)
        lse = jnp.log(jnp.sum(jnp.exp(z - mx), axis=1, keepdims=True)) + mx
        out_ref[...] = z - lse


def _k3(hc, batch3, w0, b0, w1, b1):
    return pl.pallas_call(
        _k3_body,
        grid=(NT,),
        in_specs=[
            pl.BlockSpec((4, R, 128), lambda i: (0, i, 0)),
            pl.BlockSpec((1, 1, R), lambda i: (i, 0, 0)),
            pl.BlockSpec((H, H), lambda i: (0, 0)),
            pl.BlockSpec((1, H), lambda i: (0, 0)),
            pl.BlockSpec((H, C), lambda i: (0, 0)),
            pl.BlockSpec((1, C), lambda i: (0, 0)),
        ],
        out_specs=pl.BlockSpec((G, C), lambda i: (0, 0)),
        out_shape=jax.ShapeDtypeStruct((G, C), jnp.float32),
        scratch_shapes=[
            pltpu.VMEM((4, G, 128), jnp.float32),
            pltpu.VMEM((G, 1), jnp.float32),
        ],
    )(hc, batch3, w0, b0, w1, b1)


def kernel(x, edge_index, batch,
           init_W0, init_b0, init_g0, init_be0, init_W1, init_b1, init_g1, init_be1,
           mp_W0, mp_b0, mp_g0, mp_be0, mp_W1, mp_b1, mp_g1, mp_be1,
           head_W0, head_b0, head_W1, head_b1):
    src = edge_index[0]
    dst = edge_index[1]
    x2 = jnp.transpose(x.reshape(N, 2, 128), (1, 0, 2)).reshape(2 * N, 128)
    offs2 = (jnp.arange(2, dtype=jnp.int32) * N)[:, None]
    offs4 = (jnp.arange(4, dtype=jnp.int32) * N)[:, None]
    srcm2 = (src[None, :] + offs2).reshape(2 * 16, ROWS_PER_TILE, EB)
    srcm4 = (src[None, :] + offs4).reshape(4 * 16, ROWS_PER_TILE, EB)
    dstm = dst.reshape(16, ROWS_PER_TILE, EB)
    batch3 = batch.reshape(NT, 1, R)

    s = _sc_agg(x2, srcm2, dstm, 2)
    h1, st0 = _k1(s, init_W0.reshape(2, 128, H), init_b0.reshape(1, H), 2)
    h2c, st1 = _k2(h1, st0, init_g0.reshape(1, H), init_be0.reshape(1, H),
                   init_W1, init_b1.reshape(1, H))
    xn = _k0(h2c, st1, init_g1.reshape(1, H), init_be1.reshape(1, H))
    for i in range(3):
        s = _sc_agg(xn.reshape(4 * N, 128), srcm4, dstm, 4)
        h1, st0 = _k1(s, mp_W0[i].reshape(4, 128, H), mp_b0[i].reshape(1, H), 4)
        h2c, st1 = _k2(h1, st0, mp_g0[i].reshape(1, H), mp_be0[i].reshape(1, H),
                       mp_W1[i], mp_b1[i].reshape(1, H))
        xn = _k0(h2c, st1, mp_g1[i].reshape(1, H), mp_be1[i].reshape(1, H))
    return _k3(xn, batch3, head_W0, head_b0.reshape(1, H),
               head_W1, head_b1.reshape(1, C))


# speedup vs baseline: 5.2796x; 1.5617x over previous
import functools

import jax
import jax.numpy as jnp
from jax import lax
from jax.experimental import pallas as pl
from jax.experimental.pallas import tpu as pltpu
from jax.experimental.pallas import tpu_sc as plsc

N = 10000
E = 160000
D = 256
H = 512
C = 16
G = 64
BN_EPS = 1e-5

R = 1000
NT = N // R
EB = 125
ROWS_PER_TILE = E // 16 // EB
HALF = ROWS_PER_TILE // 2
NODES_PER_TILE = N // 16


def _sc_agg_body(npc, xf, xf3, srcm, dstm, sf3, src_v, dst_v, buf0, buf1, spmem,
                 gsem0, gsem1, ssem0, ssem1):
    ci = lax.axis_index("c")
    si = lax.axis_index("s")
    slots = ((buf0, gsem0, ssem0), (buf1, gsem1, ssem1))
    pltpu.sync_copy(dstm.at[si], dst_v)
    for t in range(npc):
        chunk = ci * npc + t
        pltpu.sync_copy(xf3.at[chunk * 16 + si],
                        spmem.at[pl.ds(si * NODES_PER_TILE, NODES_PER_TILE)])
        plsc.subcore_barrier()

        for h in range(2):
            pltpu.sync_copy(srcm.at[(chunk * 16 + si) * 2 + h], src_v)
            hb = h * HALF
            pltpu.async_copy(xf.at[src_v.at[0]], buf0, gsem0)
            pltpu.async_copy(xf.at[src_v.at[1]], buf1, gsem1)

            def pair(i, carry):
                for j in range(2):
                    buf, gs, ss = slots[j]
                    b = 2 * i + j
                    pltpu.make_async_copy(xf.at[src_v.at[b]], buf, gs).wait()
                    pltpu.async_copy(buf, spmem.at[dst_v.at[hb + b]], ss,
                                     add=True)
                    pltpu.make_async_copy(buf, spmem.at[dst_v.at[hb + b]],
                                          ss).wait()

                    @pl.when(b + 2 < HALF)
                    def _():
                        pltpu.async_copy(xf.at[src_v.at[b + 2]], buf, gs)

                return carry

            lax.fori_loop(0, HALF // 2, pair, 0)

        plsc.subcore_barrier()
        pltpu.sync_copy(spmem.at[pl.ds(si * NODES_PER_TILE, NODES_PER_TILE)],
                        sf3.at[chunk * 16 + si])
        if t + 1 < npc:
            plsc.subcore_barrier()


def _sc_agg(xf, srcm, dstm, nc):
    npc = nc // 2
    mesh = plsc.VectorSubcoreMesh(core_axis_name="c", subcore_axis_name="s")
    xf3 = xf.reshape(nc * 16, NODES_PER_TILE, 128)
    out = pl.kernel(
        functools.partial(_sc_agg_body, npc),
        out_type=jax.ShapeDtypeStruct((nc * 16, NODES_PER_TILE, 128), jnp.float32),
        mesh=mesh,
        scratch_types=[
            pltpu.VMEM((HALF, EB), jnp.int32),
            pltpu.VMEM((ROWS_PER_TILE, EB), jnp.int32),
            pltpu.VMEM((EB, 128), jnp.float32),
            pltpu.VMEM((EB, 128), jnp.float32),
            pltpu.VMEM_SHARED((N, 128), jnp.float32),
            pltpu.SemaphoreType.DMA,
            pltpu.SemaphoreType.DMA,
            pltpu.SemaphoreType.DMA,
            pltpu.SemaphoreType.DMA,
        ],
    )(xf, xf3, srcm, dstm)
    return out.reshape(nc * N, 128)


def _k1_body(s_ref, w_ref, b_ref, h_ref, st_ref, acc, stacc):
    c = pl.program_id(1)
    nc = pl.num_programs(1)
    i = pl.program_id(0)
    ni = pl.num_programs(0)

    @pl.when(c == 0)
    def _():
        acc[...] = jnp.zeros_like(acc)

    acc[...] += jnp.dot(s_ref[...], w_ref[0],
                        preferred_element_type=jnp.float32)

    @pl.when(c == nc - 1)
    def _():
        h = jnp.maximum(acc[...] + b_ref[...], 0.0)
        h_ref[...] = h

        @pl.when(i == 0)
        def _():
            stacc[...] = jnp.zeros_like(stacc)

        stacc[0:1, :] += jnp.sum(h, axis=0, keepdims=True)
        stacc[1:2, :] += jnp.sum(h * h, axis=0, keepdims=True)

        @pl.when(i == ni - 1)
        def _():
            st_ref[...] = stacc[...]


def _k1(s, w, b, nc):
    return pl.pallas_call(
        _k1_body,
        grid=(NT, nc),
        in_specs=[
            pl.BlockSpec((R, 128), lambda i, c: (c * NT + i, 0)),
            pl.BlockSpec((1, 128, H), lambda i, c: (c, 0, 0)),
            pl.BlockSpec((1, H), lambda i, c: (0, 0)),
        ],
        out_specs=[
            pl.BlockSpec((R, H), lambda i, c: (i, 0)),
            pl.BlockSpec((8, H), lambda i, c: (0, 0)),
        ],
        out_shape=[
            jax.ShapeDtypeStruct((N, H), jnp.float32),
            jax.ShapeDtypeStruct((8, H), jnp.float32),
        ],
        scratch_shapes=[
            pltpu.VMEM((R, H), jnp.float32),
            pltpu.VMEM((8, H), jnp.float32),
        ],
    )(s, w, b)


def _k2_body(h_ref, st_ref, g_ref, be_ref, w_ref, b_ref, out_ref, sto_ref, stacc):
    i = pl.program_id(0)
    ni = pl.num_programs(0)
    mean = st_ref[0:1, :] * (1.0 / N)
    var = st_ref[1:2, :] * (1.0 / N) - mean * mean
    a = g_ref[...] * lax.rsqrt(var + BN_EPS)
    cc = be_ref[...] - mean * a
    hn = h_ref[...] * a + cc
    h2 = jnp.maximum(
        jnp.dot(hn, w_ref[...], preferred_element_type=jnp.float32) + b_ref[...],
        0.0)
    for c in range(4):
        out_ref[c] = h2[:, c * 128:(c + 1) * 128]

    @pl.when(i == 0)
    def _():
        stacc[...] = jnp.zeros_like(stacc)

    stacc[0:1, :] += jnp.sum(h2, axis=0, keepdims=True)
    stacc[1:2, :] += jnp.sum(h2 * h2, axis=0, keepdims=True)

    @pl.when(i == ni - 1)
    def _():
        sto_ref[...] = stacc[...]


def _k2(h, st, g, be, w, b):
    return pl.pallas_call(
        _k2_body,
        grid=(NT,),
        in_specs=[
            pl.BlockSpec((R, H), lambda i: (i, 0)),
            pl.BlockSpec((8, H), lambda i: (0, 0)),
            pl.BlockSpec((1, H), lambda i: (0, 0)),
            pl.BlockSpec((1, H), lambda i: (0, 0)),
            pl.BlockSpec((H, H), lambda i: (0, 0)),
            pl.BlockSpec((1, H), lambda i: (0, 0)),
        ],
        out_specs=[
            pl.BlockSpec((4, R, 128), lambda i: (0, i, 0)),
            pl.BlockSpec((8, H), lambda i: (0, 0)),
        ],
        out_shape=[
            jax.ShapeDtypeStruct((4, N, 128), jnp.float32),
            jax.ShapeDtypeStruct((8, H), jnp.float32),
        ],
        scratch_shapes=[pltpu.VMEM((8, H), jnp.float32)],
    )(h, st, g, be, w, b)


def _k0_body(h_ref, st_ref, g_ref, be_ref, out_ref):
    mean = st_ref[0:1, :] * (1.0 / N)
    var = st_ref[1:2, :] * (1.0 / N) - mean * mean
    a = g_ref[...] * lax.rsqrt(var + BN_EPS)
    cc = be_ref[...] - mean * a
    for c in range(4):
        sl = slice(c * 128, (c + 1) * 128)
        out_ref[c] = h_ref[c] * a[:, sl] + cc[:, sl]


def _k0(hc, st, g, be):
    return pl.pallas_call(
        _k0_body,
        grid=(NT,),
        in_specs=[
            pl.BlockSpec((4, R, 128), lambda i: (0, i, 0)),
            pl.BlockSpec((8, H), lambda i: (0, 0)),
            pl.BlockSpec((1, H), lambda i: (0, 0)),
            pl.BlockSpec((1, H), lambda i: (0, 0)),
        ],
        out_specs=pl.BlockSpec((4, R, 128), lambda i: (0, i, 0)),
        out_shape=jax.ShapeDtypeStruct((4, N, 128), jnp.float32),
    )(hc, st, g, be)


def _k3_body(h_ref, b_ref, w0_ref, b0_ref, w1_ref, b1_ref, out_ref, psum, cnt):
    i = pl.program_id(0)
    ni = pl.num_programs(0)

    @pl.when(i == 0)
    def _():
        psum[...] = jnp.zeros_like(psum)
        cnt[...] = jnp.zeros_like(cnt)

    bt = b_ref[0]
    gids = lax.broadcasted_iota(jnp.int32, (G, 1), 0)
    oh = (bt == gids).astype(jnp.float32)
    for c in range(4):
        psum[c] += jnp.dot(oh, h_ref[c], preferred_element_type=jnp.float32)
    cnt[...] += jnp.sum(oh, axis=1, keepdims=True)

    @pl.when(i == ni - 1)
    def _():
        inv = 1.0 / jnp.maximum(cnt[...], 1.0)
        z1 = b0_ref[...] + jnp.zeros((G, H), jnp.float32)
        for c in range(4):
            z1 += jnp.dot(psum[c] * inv, w0_ref[c * 128:(c + 1) * 128, :],
                          preferred_element_type=jnp.float32)
        z1 = jnp.maximum(z1, 0.0)
        z = jnp.dot(z1, w1_ref[...], preferred_element_type=jnp.float32) + b1_ref[...]
        mx = jnp.max(z, axis=1, keepdims=True)
        lse = jnp.log(jnp.sum(jnp.exp(z - mx), axis=1, keepdims=True)) + mx
        out_ref[...] = z - lse


def _k3(hc, batch3, w0, b0, w1, b1):
    return pl.pallas_call(
        _k3_body,
        grid=(NT,),
        in_specs=[
            pl.BlockSpec((4, R, 128), lambda i: (0, i, 0)),
            pl.BlockSpec((1, 1, R), lambda i: (i, 0, 0)),
            pl.BlockSpec((H, H), lambda i: (0, 0)),
            pl.BlockSpec((1, H), lambda i: (0, 0)),
            pl.BlockSpec((H, C), lambda i: (0, 0)),
            pl.BlockSpec((1, C), lambda i: (0, 0)),
        ],
        out_specs=pl.BlockSpec((G, C), lambda i: (0, 0)),
        out_shape=jax.ShapeDtypeStruct((G, C), jnp.float32),
        scratch_shapes=[
            pltpu.VMEM((4, G, 128), jnp.float32),
            pltpu.VMEM((G, 1), jnp.float32),
        ],
    )(hc, batch3, w0, b0, w1, b1)


def kernel(x, edge_index, batch,
           init_W0, init_b0, init_g0, init_be0, init_W1, init_b1, init_g1, init_be1,
           mp_W0, mp_b0, mp_g0, mp_be0, mp_W1, mp_b1, mp_g1, mp_be1,
           head_W0, head_b0, head_W1, head_b1):
    src = edge_index[0]
    dst = edge_index[1]
    x2 = jnp.transpose(x.reshape(N, 2, 128), (1, 0, 2)).reshape(2 * N, 128)
    offs2 = (jnp.arange(2, dtype=jnp.int32) * N)[:, None]
    offs4 = (jnp.arange(4, dtype=jnp.int32) * N)[:, None]
    srcm2 = (src[None, :] + offs2).reshape(2 * 32, HALF, EB)
    srcm4 = (src[None, :] + offs4).reshape(4 * 32, HALF, EB)
    dstm = dst.reshape(16, ROWS_PER_TILE, EB)
    batch3 = batch.reshape(NT, 1, R)

    s = _sc_agg(x2, srcm2, dstm, 2)
    h1, st0 = _k1(s, init_W0.reshape(2, 128, H), init_b0.reshape(1, H), 2)
    h2c, st1 = _k2(h1, st0, init_g0.reshape(1, H), init_be0.reshape(1, H),
                   init_W1, init_b1.reshape(1, H))
    xn = _k0(h2c, st1, init_g1.reshape(1, H), init_be1.reshape(1, H))
    for i in range(3):
        s = _sc_agg(xn.reshape(4 * N, 128), srcm4, dstm, 4)
        h1, st0 = _k1(s, mp_W0[i].reshape(4, 128, H), mp_b0[i].reshape(1, H), 4)
        h2c, st1 = _k2(h1, st0, mp_g0[i].reshape(1, H), mp_be0[i].reshape(1, H),
                       mp_W1[i], mp_b1[i].reshape(1, H))
        xn = _k0(h2c, st1, mp_g1[i].reshape(1, H), mp_be1[i].reshape(1, H))
    return _k3(xn, batch3, head_W0, head_b0.reshape(1, H),
               head_W1, head_b1.reshape(1, C))


# speedup vs baseline: 5.4472x; 1.0317x over previous
import functools

import jax
import jax.numpy as jnp
from jax import lax
from jax.experimental import pallas as pl
from jax.experimental.pallas import tpu as pltpu
from jax.experimental.pallas import tpu_sc as plsc

N = 10000
E = 160000
D = 256
H = 512
C = 16
G = 64
BN_EPS = 1e-5

R = 1000
NT = N // R
EB = 125
ROWS_PER_TILE = E // 16 // EB
HALF = ROWS_PER_TILE // 2
NODES_PER_TILE = N // 16
BF = jnp.bfloat16


def _bdot(a, b):
    return jnp.dot(a.astype(BF), b.astype(BF),
                   preferred_element_type=jnp.float32)


def _sc_agg_body(npc, xf, xf3, srcm, dstm, sf3, src_v, dst_v, buf0, buf1, spmem,
                 gsem0, gsem1, ssem0, ssem1):
    ci = lax.axis_index("c")
    si = lax.axis_index("s")
    slots = ((buf0, gsem0, ssem0), (buf1, gsem1, ssem1))
    pltpu.sync_copy(dstm.at[si], dst_v)
    for t in range(npc):
        chunk = ci * npc + t
        pltpu.sync_copy(xf3.at[chunk * 16 + si],
                        spmem.at[pl.ds(si * NODES_PER_TILE, NODES_PER_TILE)])
        plsc.subcore_barrier()

        for h in range(2):
            pltpu.sync_copy(srcm.at[(chunk * 16 + si) * 2 + h], src_v)
            hb = h * HALF
            pltpu.async_copy(xf.at[src_v.at[0]], buf0, gsem0)
            pltpu.async_copy(xf.at[src_v.at[1]], buf1, gsem1)

            def pair(i, carry):
                for j in range(2):
                    buf, gs, ss = slots[j]
                    b = 2 * i + j
                    pltpu.make_async_copy(xf.at[src_v.at[b]], buf, gs).wait()
                    pltpu.async_copy(buf, spmem.at[dst_v.at[hb + b]], ss,
                                     add=True)
                    pltpu.make_async_copy(buf, spmem.at[dst_v.at[hb + b]],
                                          ss).wait()

                    @pl.when(b + 2 < HALF)
                    def _():
                        pltpu.async_copy(xf.at[src_v.at[b + 2]], buf, gs)

                return carry

            lax.fori_loop(0, HALF // 2, pair, 0)

        plsc.subcore_barrier()
        pltpu.sync_copy(spmem.at[pl.ds(si * NODES_PER_TILE, NODES_PER_TILE)],
                        sf3.at[chunk * 16 + si])
        if t + 1 < npc:
            plsc.subcore_barrier()


def _sc_agg(xf, srcm, dstm, nc):
    npc = nc // 2
    mesh = plsc.VectorSubcoreMesh(core_axis_name="c", subcore_axis_name="s")
    xf3 = xf.reshape(nc * 16, NODES_PER_TILE, 128)
    out = pl.kernel(
        functools.partial(_sc_agg_body, npc),
        out_type=jax.ShapeDtypeStruct((nc * 16, NODES_PER_TILE, 128),
                                      jnp.float32),
        mesh=mesh,
        scratch_types=[
            pltpu.VMEM((HALF, EB), jnp.int32),
            pltpu.VMEM((ROWS_PER_TILE, EB), jnp.int32),
            pltpu.VMEM((EB, 128), jnp.float32),
            pltpu.VMEM((EB, 128), jnp.float32),
            pltpu.VMEM_SHARED((N, 128), jnp.float32),
            pltpu.SemaphoreType.DMA,
            pltpu.SemaphoreType.DMA,
            pltpu.SemaphoreType.DMA,
            pltpu.SemaphoreType.DMA,
        ],
    )(xf, xf3, srcm, dstm)
    return out.reshape(nc * N, 128)


def _k1_body(s_ref, w_ref, b_ref, h_ref, st_ref, acc, stacc):
    c = pl.program_id(1)
    nc = pl.num_programs(1)
    i = pl.program_id(0)
    ni = pl.num_programs(0)

    @pl.when(c == 0)
    def _():
        acc[...] = jnp.zeros_like(acc)

    acc[...] += _bdot(s_ref[...], w_ref[0])

    @pl.when(c == nc - 1)
    def _():
        h = jnp.maximum(acc[...] + b_ref[...], 0.0)
        h_ref[...] = h.astype(BF)

        @pl.when(i == 0)
        def _():
            stacc[...] = jnp.zeros_like(stacc)

        stacc[0:1, :] += jnp.sum(h, axis=0, keepdims=True)
        stacc[1:2, :] += jnp.sum(h * h, axis=0, keepdims=True)

        @pl.when(i == ni - 1)
        def _():
            st_ref[...] = stacc[...]


def _k1(s, w, b, nc):
    return pl.pallas_call(
        _k1_body,
        grid=(NT, nc),
        in_specs=[
            pl.BlockSpec((R, 128), lambda i, c: (c * NT + i, 0)),
            pl.BlockSpec((1, 128, H), lambda i, c: (c, 0, 0)),
            pl.BlockSpec((1, H), lambda i, c: (0, 0)),
        ],
        out_specs=[
            pl.BlockSpec((R, H), lambda i, c: (i, 0)),
            pl.BlockSpec((8, H), lambda i, c: (0, 0)),
        ],
        out_shape=[
            jax.ShapeDtypeStruct((N, H), BF),
            jax.ShapeDtypeStruct((8, H), jnp.float32),
        ],
        scratch_shapes=[
            pltpu.VMEM((R, H), jnp.float32),
            pltpu.VMEM((8, H), jnp.float32),
        ],
    )(s, w, b)


def _k2_body(h_ref, st_ref, g_ref, be_ref, w_ref, b_ref, out_ref, sto_ref, stacc):
    i = pl.program_id(0)
    ni = pl.num_programs(0)
    mean = st_ref[0:1, :] * (1.0 / N)
    var = st_ref[1:2, :] * (1.0 / N) - mean * mean
    a = g_ref[...] * lax.rsqrt(var + BN_EPS)
    cc = be_ref[...] - mean * a
    hn = h_ref[...].astype(jnp.float32) * a + cc
    h2 = jnp.maximum(_bdot(hn, w_ref[...]) + b_ref[...], 0.0)
    out_ref[...] = h2.astype(BF)

    @pl.when(i == 0)
    def _():
        stacc[...] = jnp.zeros_like(stacc)

    stacc[0:1, :] += jnp.sum(h2, axis=0, keepdims=True)
    stacc[1:2, :] += jnp.sum(h2 * h2, axis=0, keepdims=True)

    @pl.when(i == ni - 1)
    def _():
        sto_ref[...] = stacc[...]


def _k2(h, st, g, be, w, b):
    return pl.pallas_call(
        _k2_body,
        grid=(NT,),
        in_specs=[
            pl.BlockSpec((R, H), lambda i: (i, 0)),
            pl.BlockSpec((8, H), lambda i: (0, 0)),
            pl.BlockSpec((1, H), lambda i: (0, 0)),
            pl.BlockSpec((1, H), lambda i: (0, 0)),
            pl.BlockSpec((H, H), lambda i: (0, 0)),
            pl.BlockSpec((1, H), lambda i: (0, 0)),
        ],
        out_specs=[
            pl.BlockSpec((R, H), lambda i: (i, 0)),
            pl.BlockSpec((8, H), lambda i: (0, 0)),
        ],
        out_shape=[
            jax.ShapeDtypeStruct((N, H), BF),
            jax.ShapeDtypeStruct((8, H), jnp.float32),
        ],
        scratch_shapes=[pltpu.VMEM((8, H), jnp.float32)],
    )(h, st, g, be, w, b)


def _k0_body(h_ref, st_ref, g_ref, be_ref, out_ref):
    mean = st_ref[0:1, :] * (1.0 / N)
    var = st_ref[1:2, :] * (1.0 / N) - mean * mean
    a = g_ref[...] * lax.rsqrt(var + BN_EPS)
    cc = be_ref[...] - mean * a
    hn = h_ref[...].astype(jnp.float32) * a + cc
    for c in range(4):
        out_ref[c] = hn[:, c * 128:(c + 1) * 128]


def _k0(hc, st, g, be):
    return pl.pallas_call(
        _k0_body,
        grid=(NT,),
        in_specs=[
            pl.BlockSpec((R, H), lambda i: (i, 0)),
            pl.BlockSpec((8, H), lambda i: (0, 0)),
            pl.BlockSpec((1, H), lambda i: (0, 0)),
            pl.BlockSpec((1, H), lambda i: (0, 0)),
        ],
        out_specs=pl.BlockSpec((4, R, 128), lambda i: (0, i, 0)),
        out_shape=jax.ShapeDtypeStruct((4, N, 128), jnp.float32),
    )(hc, st, g, be)


def _k3_body(h_ref, b_ref, w0_ref, b0_ref, w1_ref, b1_ref, out_ref, psum, cnt):
    i = pl.program_id(0)
    ni = pl.num_programs(0)

    @pl.when(i == 0)
    def _():
        psum[...] = jnp.zeros_like(psum)
        cnt[...] = jnp.zeros_like(cnt)

    bt = b_ref[0]
    gids = lax.broadcasted_iota(jnp.int32, (G, 1), 0)
    oh = (bt == gids).astype(jnp.float32)
    for c in range(4):
        psum[c] += _bdot(oh, h_ref[c])
    cnt[...] += jnp.sum(oh, axis=1, keepdims=True)

    @pl.when(i == ni - 1)
    def _():
        inv = 1.0 / jnp.maximum(cnt[...], 1.0)
        z1 = b0_ref[...] + jnp.zeros((G, H), jnp.float32)
        for c in range(4):
            z1 += _bdot(psum[c] * inv, w0_ref[c * 128:(c + 1) * 128, :])
        z1 = jnp.maximum(z1, 0.0)
        z = _bdot(z1, w1_ref[...]) + b1_ref[...]
        mx = jnp.max(z, axis=1, keepdims=True)
        lse = jnp.log(jnp.sum(jnp.exp(z - mx), axis=1, keepdims=True)) + mx
        out_ref[...] = z - lse


def _k3(hc, batch3, w0, b0, w1, b1):
    return pl.pallas_call(
        _k3_body,
        grid=(NT,),
        in_specs=[
            pl.BlockSpec((4, R, 128), lambda i: (0, i, 0)),
            pl.BlockSpec((1, 1, R), lambda i: (i, 0, 0)),
            pl.BlockSpec((H, H), lambda i: (0, 0)),
            pl.BlockSpec((1, H), lambda i: (0, 0)),
            pl.BlockSpec((H, C), lambda i: (0, 0)),
            pl.BlockSpec((1, C), lambda i: (0, 0)),
        ],
        out_specs=pl.BlockSpec((G, C), lambda i: (0, 0)),
        out_shape=jax.ShapeDtypeStruct((G, C), jnp.float32),
        scratch_shapes=[
            pltpu.VMEM((4, G, 128), jnp.float32),
            pltpu.VMEM((G, 1), jnp.float32),
        ],
    )(hc, batch3, w0, b0, w1, b1)


def kernel(x, edge_index, batch,
           init_W0, init_b0, init_g0, init_be0, init_W1, init_b1, init_g1, init_be1,
           mp_W0, mp_b0, mp_g0, mp_be0, mp_W1, mp_b1, mp_g1, mp_be1,
           head_W0, head_b0, head_W1, head_b1):
    src = edge_index[0]
    dst = edge_index[1]
    x2 = jnp.transpose(x.reshape(N, 2, 128), (1, 0, 2)).reshape(2 * N, 128)
    offs2 = (jnp.arange(2, dtype=jnp.int32) * N)[:, None]
    offs4 = (jnp.arange(4, dtype=jnp.int32) * N)[:, None]
    srcm2 = (src[None, :] + offs2).reshape(2 * 32, HALF, EB)
    srcm4 = (src[None, :] + offs4).reshape(4 * 32, HALF, EB)
    dstm = dst.reshape(16, ROWS_PER_TILE, EB)
    batch3 = batch.reshape(NT, 1, R)

    s = _sc_agg(x2, srcm2, dstm, 2)
    h1, st0 = _k1(s, init_W0.reshape(2, 128, H), init_b0.reshape(1, H), 2)
    h2c, st1 = _k2(h1, st0, init_g0.reshape(1, H), init_be0.reshape(1, H),
                   init_W1, init_b1.reshape(1, H))
    xn = _k0(h2c, st1, init_g1.reshape(1, H), init_be1.reshape(1, H))
    for i in range(3):
        s = _sc_agg(xn.reshape(4 * N, 128), srcm4, dstm, 4)
        h1, st0 = _k1(s, mp_W0[i].reshape(4, 128, H), mp_b0[i].reshape(1, H), 4)
        h2c, st1 = _k2(h1, st0, mp_g0[i].reshape(1, H), mp_be0[i].reshape(1, H),
                       mp_W1[i], mp_b1[i].reshape(1, H))
        xn = _k0(h2c, st1, mp_g1[i].reshape(1, H), mp_be1[i].reshape(1, H))
    return _k3(xn, batch3, head_W0, head_b0.reshape(1, H),
               head_W1, head_b1.reshape(1, C))


# speedup vs baseline: 5.6057x; 1.0291x over previous
import functools

import jax
import jax.numpy as jnp
from jax import lax
from jax.experimental import pallas as pl
from jax.experimental.pallas import tpu as pltpu
from jax.experimental.pallas import tpu_sc as plsc

N = 10000
E = 160000
D = 256
H = 512
C = 16
G = 64
BN_EPS = 1e-5

R = 1000
NT = N // R
EB = 125
ROWS_PER_TILE = E // 16 // EB
HALF = ROWS_PER_TILE // 2
NODES_PER_TILE = N // 16
BF = jnp.bfloat16


def _bdot(a, b):
    return jnp.dot(a.astype(BF), b.astype(BF),
                   preferred_element_type=jnp.float32)


def _sc_agg_body(npc, xf, xf3, srcm, dstm, sf3, src_v, dst_v, buf0, buf1, spmem,
                 gsem0, gsem1, ssem0, ssem1):
    ci = lax.axis_index("c")
    si = lax.axis_index("s")
    slots = ((buf0, gsem0, ssem0), (buf1, gsem1, ssem1))
    pltpu.sync_copy(dstm.at[si], dst_v)
    for t in range(npc):
        chunk = ci * npc + t
        pltpu.sync_copy(xf3.at[chunk * 16 + si],
                        spmem.at[pl.ds(si * NODES_PER_TILE, NODES_PER_TILE)])
        plsc.subcore_barrier()

        for h in range(2):
            pltpu.sync_copy(srcm.at[(chunk * 16 + si) * 2 + h], src_v)
            hb = h * HALF
            pltpu.async_copy(xf.at[src_v.at[0]], buf0, gsem0)
            pltpu.async_copy(xf.at[src_v.at[1]], buf1, gsem1)

            def pair(i, carry):
                for j in range(2):
                    buf, gs, ss = slots[j]
                    b = 2 * i + j
                    pltpu.make_async_copy(xf.at[src_v.at[b]], buf, gs).wait()
                    pltpu.async_copy(buf, spmem.at[dst_v.at[hb + b]], ss,
                                     add=True)
                    pltpu.make_async_copy(buf, spmem.at[dst_v.at[hb + b]],
                                          ss).wait()

                    @pl.when(b + 2 < HALF)
                    def _():
                        pltpu.async_copy(xf.at[src_v.at[b + 2]], buf, gs)

                return carry

            lax.fori_loop(0, HALF // 2, pair, 0)

        plsc.subcore_barrier()
        pltpu.sync_copy(spmem.at[pl.ds(si * NODES_PER_TILE, NODES_PER_TILE)],
                        sf3.at[chunk * 16 + si])
        if t + 1 < npc:
            plsc.subcore_barrier()


def _sc_agg(xf, srcm, dstm, nc):
    npc = nc // 2
    mesh = plsc.VectorSubcoreMesh(core_axis_name="c", subcore_axis_name="s")
    xf3 = xf.reshape(nc * 16, NODES_PER_TILE, 128)
    out = pl.kernel(
        functools.partial(_sc_agg_body, npc),
        out_type=jax.ShapeDtypeStruct((nc * 16, NODES_PER_TILE, 128),
                                      jnp.float32),
        mesh=mesh,
        scratch_types=[
            pltpu.VMEM((HALF, EB), jnp.int32),
            pltpu.VMEM((ROWS_PER_TILE, EB), jnp.int32),
            pltpu.VMEM((EB, 128), jnp.float32),
            pltpu.VMEM((EB, 128), jnp.float32),
            pltpu.VMEM_SHARED((N, 128), jnp.float32),
            pltpu.SemaphoreType.DMA,
            pltpu.SemaphoreType.DMA,
            pltpu.SemaphoreType.DMA,
            pltpu.SemaphoreType.DMA,
        ],
    )(xf, xf3, srcm, dstm)
    return out.reshape(nc * N, 128)


def _sc_deg_body(dstm, deg_out, dst_v, zbuf, spdeg, s0, s1):
    ci = lax.axis_index("c")
    si = lax.axis_index("s")
    pltpu.sync_copy(dstm.at[si], dst_v)

    def fill(val):
        def body(r, c):
            for k in range(8):
                zbuf[r, pl.ds(k * 16, 16)] = jnp.full((16,), val, jnp.float32)
            return c
        lax.fori_loop(0, EB, body, 0)

    fill(0.0)
    for k in range(5):
        pltpu.sync_copy(zbuf, spdeg.at[pl.ds(si * NODES_PER_TILE + k * EB, EB)])
    fill(1.0)
    plsc.subcore_barrier()
    bo = ci * HALF
    pltpu.async_copy(zbuf, spdeg.at[dst_v.at[bo]], s0, add=True)
    pltpu.async_copy(zbuf, spdeg.at[dst_v.at[bo + 1]], s1, add=True)

    def pair(i, carry):
        b = bo + 2 * i
        for j, sem in ((0, s0), (1, s1)):
            pltpu.make_async_copy(zbuf, spdeg.at[dst_v.at[b + j]], sem).wait()

            @pl.when(b + j + 2 < bo + HALF)
            def _():
                pltpu.async_copy(zbuf, spdeg.at[dst_v.at[b + j + 2]], sem,
                                 add=True)

        return carry

    lax.fori_loop(0, HALF // 2, pair, 0)
    plsc.subcore_barrier()
    pltpu.sync_copy(spdeg.at[pl.ds(si * NODES_PER_TILE, NODES_PER_TILE)],
                    deg_out.at[ci * 16 + si])


def _sc_deg(dstm):
    mesh = plsc.VectorSubcoreMesh(core_axis_name="c", subcore_axis_name="s")
    out = pl.kernel(
        _sc_deg_body,
        out_type=jax.ShapeDtypeStruct((32, NODES_PER_TILE, 128), jnp.float32),
        mesh=mesh,
        scratch_types=[
            pltpu.VMEM((ROWS_PER_TILE, EB), jnp.int32),
            pltpu.VMEM((EB, 128), jnp.float32),
            pltpu.VMEM_SHARED((N, 128), jnp.float32),
            pltpu.SemaphoreType.DMA,
            pltpu.SemaphoreType.DMA,
        ],
    )(dstm)
    return out.reshape(2, N, 128)


def _k1_body(s_ref, w_ref, b_ref, h_ref, st_ref, acc, stacc):
    c = pl.program_id(1)
    nc = pl.num_programs(1)
    i = pl.program_id(0)
    ni = pl.num_programs(0)

    @pl.when(c == 0)
    def _():
        acc[...] = jnp.zeros_like(acc)

    acc[...] += _bdot(s_ref[...], w_ref[0])

    @pl.when(c == nc - 1)
    def _():
        h = jnp.maximum(acc[...] + b_ref[...], 0.0)
        h_ref[...] = h.astype(BF)

        @pl.when(i == 0)
        def _():
            stacc[...] = jnp.zeros_like(stacc)

        stacc[0:1, :] += jnp.sum(h, axis=0, keepdims=True)
        stacc[1:2, :] += jnp.sum(h * h, axis=0, keepdims=True)

        @pl.when(i == ni - 1)
        def _():
            st_ref[...] = stacc[...]


def _k1(s, w, b, nc):
    return pl.pallas_call(
        _k1_body,
        grid=(NT, nc),
        in_specs=[
            pl.BlockSpec((R, 128), lambda i, c: (c * NT + i, 0)),
            pl.BlockSpec((1, 128, H), lambda i, c: (c, 0, 0)),
            pl.BlockSpec((1, H), lambda i, c: (0, 0)),
        ],
        out_specs=[
            pl.BlockSpec((R, H), lambda i, c: (i, 0)),
            pl.BlockSpec((8, H), lambda i, c: (0, 0)),
        ],
        out_shape=[
            jax.ShapeDtypeStruct((N, H), BF),
            jax.ShapeDtypeStruct((8, H), jnp.float32),
        ],
        scratch_shapes=[
            pltpu.VMEM((R, H), jnp.float32),
            pltpu.VMEM((8, H), jnp.float32),
        ],
    )(s, w, b)


def _k1n_body(s_ref, w_ref, b_ref, stp_ref, gp_ref, bep_ref, deg_ref,
              h_ref, st_ref, acc, vacc, stacc):
    c = pl.program_id(1)
    nc = pl.num_programs(1)
    i = pl.program_id(0)
    ni = pl.num_programs(0)
    mean = stp_ref[0, 0:1, :] * (1.0 / N)
    var = stp_ref[0, 1:2, :] * (1.0 / N) - mean * mean
    a = gp_ref[0] * lax.rsqrt(var + BN_EPS)
    cc = bep_ref[0] - mean * a

    @pl.when(c == 0)
    def _():
        acc[...] = jnp.zeros_like(acc)
        vacc[...] = jnp.zeros_like(vacc)

    acc[...] += _bdot(s_ref[...] * a, w_ref[0])
    vacc[...] += _bdot(cc, w_ref[0])

    @pl.when(c == nc - 1)
    def _():
        degc = deg_ref[0, :, 0:1] + deg_ref[1, :, 0:1] + 1.0
        h = jnp.maximum(acc[...] + degc * vacc[...] + b_ref[...], 0.0)
        h_ref[...] = h.astype(BF)

        @pl.when(i == 0)
        def _():
            stacc[...] = jnp.zeros_like(stacc)

        stacc[0:1, :] += jnp.sum(h, axis=0, keepdims=True)
        stacc[1:2, :] += jnp.sum(h * h, axis=0, keepdims=True)

        @pl.when(i == ni - 1)
        def _():
            st_ref[...] = stacc[...]


def _k1n(s, w, b, stp, gp, bep, deg, nc):
    return pl.pallas_call(
        _k1n_body,
        grid=(NT, nc),
        in_specs=[
            pl.BlockSpec((R, 128), lambda i, c: (c * NT + i, 0)),
            pl.BlockSpec((1, 128, H), lambda i, c: (c, 0, 0)),
            pl.BlockSpec((1, H), lambda i, c: (0, 0)),
            pl.BlockSpec((1, 8, 128), lambda i, c: (c, 0, 0)),
            pl.BlockSpec((1, 1, 128), lambda i, c: (c, 0, 0)),
            pl.BlockSpec((1, 1, 128), lambda i, c: (c, 0, 0)),
            pl.BlockSpec((2, R, 128), lambda i, c: (0, i, 0)),
        ],
        out_specs=[
            pl.BlockSpec((R, H), lambda i, c: (i, 0)),
            pl.BlockSpec((8, H), lambda i, c: (0, 0)),
        ],
        out_shape=[
            jax.ShapeDtypeStruct((N, H), BF),
            jax.ShapeDtypeStruct((8, H), jnp.float32),
        ],
        scratch_shapes=[
            pltpu.VMEM((R, H), jnp.float32),
            pltpu.VMEM((1, H), jnp.float32),
            pltpu.VMEM((8, H), jnp.float32),
        ],
    )(s, w, b, jnp.transpose(stp.reshape(8, nc, 128), (1, 0, 2)),
      gp.reshape(nc, 1, 128), bep.reshape(nc, 1, 128), deg)


def _k2_body(h_ref, st_ref, g_ref, be_ref, w_ref, b_ref, out_ref, sto_ref, stacc):
    i = pl.program_id(0)
    ni = pl.num_programs(0)
    mean = st_ref[0:1, :] * (1.0 / N)
    var = st_ref[1:2, :] * (1.0 / N) - mean * mean
    a = g_ref[...] * lax.rsqrt(var + BN_EPS)
    cc = be_ref[...] - mean * a
    hn = h_ref[...].astype(jnp.float32) * a + cc
    h2 = jnp.maximum(_bdot(hn, w_ref[...]) + b_ref[...], 0.0)
    for c in range(4):
        out_ref[c] = h2[:, c * 128:(c + 1) * 128]

    @pl.when(i == 0)
    def _():
        stacc[...] = jnp.zeros_like(stacc)

    stacc[0:1, :] += jnp.sum(h2, axis=0, keepdims=True)
    stacc[1:2, :] += jnp.sum(h2 * h2, axis=0, keepdims=True)

    @pl.when(i == ni - 1)
    def _():
        sto_ref[...] = stacc[...]


def _k2(h, st, g, be, w, b):
    return pl.pallas_call(
        _k2_body,
        grid=(NT,),
        in_specs=[
            pl.BlockSpec((R, H), lambda i: (i, 0)),
            pl.BlockSpec((8, H), lambda i: (0, 0)),
            pl.BlockSpec((1, H), lambda i: (0, 0)),
            pl.BlockSpec((1, H), lambda i: (0, 0)),
            pl.BlockSpec((H, H), lambda i: (0, 0)),
            pl.BlockSpec((1, H), lambda i: (0, 0)),
        ],
        out_specs=[
            pl.BlockSpec((4, R, 128), lambda i: (0, i, 0)),
            pl.BlockSpec((8, H), lambda i: (0, 0)),
        ],
        out_shape=[
            jax.ShapeDtypeStruct((4, N, 128), jnp.float32),
            jax.ShapeDtypeStruct((8, H), jnp.float32),
        ],
        scratch_shapes=[pltpu.VMEM((8, H), jnp.float32)],
    )(h, st, g, be, w, b)


def _k3_body(h_ref, b_ref, st_ref, g_ref, be_ref, w0_ref, b0_ref, w1_ref,
             b1_ref, out_ref, psum, cnt):
    i = pl.program_id(0)
    ni = pl.num_programs(0)

    @pl.when(i == 0)
    def _():
        psum[...] = jnp.zeros_like(psum)
        cnt[...] = jnp.zeros_like(cnt)

    bt = b_ref[0]
    gids = lax.broadcasted_iota(jnp.int32, (G, 1), 0)
    oh = (bt == gids).astype(jnp.float32)
    for c in range(4):
        psum[c] += _bdot(oh, h_ref[c])
    cnt[...] += jnp.sum(oh, axis=1, keepdims=True)

    @pl.when(i == ni - 1)
    def _():
        mean = st_ref[0:1, :] * (1.0 / N)
        var = st_ref[1:2, :] * (1.0 / N) - mean * mean
        a = g_ref[...] * lax.rsqrt(var + BN_EPS)
        cc = be_ref[...] - mean * a
        inv = 1.0 / jnp.maximum(cnt[...], 1.0)
        z1 = b0_ref[...] + jnp.zeros((G, H), jnp.float32)
        for c in range(4):
            sl = slice(c * 128, (c + 1) * 128)
            pn = psum[c] * inv * a[:, sl] + cc[:, sl]
            z1 += _bdot(pn, w0_ref[sl, :])
        z1 = jnp.maximum(z1, 0.0)
        z = _bdot(z1, w1_ref[...]) + b1_ref[...]
        mx = jnp.max(z, axis=1, keepdims=True)
        lse = jnp.log(jnp.sum(jnp.exp(z - mx), axis=1, keepdims=True)) + mx
        out_ref[...] = z - lse


def _k3(hc, batch3, st, g, be, w0, b0, w1, b1):
    return pl.pallas_call(
        _k3_body,
        grid=(NT,),
        in_specs=[
            pl.BlockSpec((4, R, 128), lambda i: (0, i, 0)),
            pl.BlockSpec((1, 1, R), lambda i: (i, 0, 0)),
            pl.BlockSpec((8, H), lambda i: (0, 0)),
            pl.BlockSpec((1, H), lambda i: (0, 0)),
            pl.BlockSpec((1, H), lambda i: (0, 0)),
            pl.BlockSpec((H, H), lambda i: (0, 0)),
            pl.BlockSpec((1, H), lambda i: (0, 0)),
            pl.BlockSpec((H, C), lambda i: (0, 0)),
            pl.BlockSpec((1, C), lambda i: (0, 0)),
        ],
        out_specs=pl.BlockSpec((G, C), lambda i: (0, 0)),
        out_shape=jax.ShapeDtypeStruct((G, C), jnp.float32),
        scratch_shapes=[
            pltpu.VMEM((4, G, 128), jnp.float32),
            pltpu.VMEM((G, 1), jnp.float32),
        ],
    )(hc, batch3, st, g, be, w0, b0, w1, b1)


def kernel(x, edge_index, batch,
           init_W0, init_b0, init_g0, init_be0, init_W1, init_b1, init_g1, init_be1,
           mp_W0, mp_b0, mp_g0, mp_be0, mp_W1, mp_b1, mp_g1, mp_be1,
           head_W0, head_b0, head_W1, head_b1):
    src = edge_index[0]
    dst = edge_index[1]
    x2 = jnp.transpose(x.reshape(N, 2, 128), (1, 0, 2)).reshape(2 * N, 128)
    offs2 = (jnp.arange(2, dtype=jnp.int32) * N)[:, None]
    offs4 = (jnp.arange(4, dtype=jnp.int32) * N)[:, None]
    srcm2 = (src[None, :] + offs2).reshape(2 * 32, HALF, EB)
    srcm4 = (src[None, :] + offs4).reshape(4 * 32, HALF, EB)
    dstm = dst.reshape(16, ROWS_PER_TILE, EB)
    batch3 = batch.reshape(NT, 1, R)

    deg = _sc_deg(dstm)
    s = _sc_agg(x2, srcm2, dstm, 2)
    h1, st0 = _k1(s, init_W0.reshape(2, 128, H), init_b0.reshape(1, H), 2)
    h2c, st1 = _k2(h1, st0, init_g0.reshape(1, H), init_be0.reshape(1, H),
                   init_W1, init_b1.reshape(1, H))
    gp, bep = init_g1.reshape(1, H), init_be1.reshape(1, H)
    for i in range(3):
        s = _sc_agg(h2c.reshape(4 * N, 128), srcm4, dstm, 4)
        h1, st0 = _k1n(s, mp_W0[i].reshape(4, 128, H), mp_b0[i].reshape(1, H),
                       st1, gp, bep, deg, 4)
        h2c, st1 = _k2(h1, st0, mp_g0[i].reshape(1, H), mp_be0[i].reshape(1, H),
                       mp_W1[i], mp_b1[i].reshape(1, H))
        gp, bep = mp_g1[i].reshape(1, H), mp_be1[i].reshape(1, H)
    return _k3(h2c, batch3, st1, gp, bep, head_W0, head_b0.reshape(1, H),
               head_W1, head_b1.reshape(1, C))
